# Initial kernel scaffold; baseline (speedup 1.0000x reference)
import jax
import jax.numpy as jnp
from jax.experimental import pallas as pl


def kernel(x, select_num):
    raise NotImplementedError("write your pallas kernel here")


# speedup vs baseline: 2.3952x; 2.3952x over previous
import jax
import jax.numpy as jnp
from jax import lax
from jax.experimental import pallas as pl

B = 32
NUM_HEADS = 12
PATCH_NUM = 576
VOTE_PERHEAD = 24
SELECT_NUM = 128
GRID_W = 24
BPG = 8


def _body(s_ref, o_ref):
    s = s_ref[...]
    iota = lax.broadcasted_iota(jnp.int32, (NUM_HEADS, BPG, PATCH_NUM), 2)
    cnt = jnp.zeros((NUM_HEADS, BPG, PATCH_NUM), jnp.float32)
    for _ in range(VOTE_PERHEAD):
        m = jnp.max(s, axis=2, keepdims=True)
        idx = jnp.min(jnp.where(s == m, iota, PATCH_NUM), axis=2, keepdims=True)
        oh = iota == idx
        cnt = jnp.where(oh, cnt + 1.0, cnt)
        s = jnp.where(oh, -jnp.inf, s)
    count = jnp.sum(cnt, axis=0)

    col = lax.broadcasted_iota(jnp.int32, (BPG, PATCH_NUM), 1) % GRID_W
    z1 = jnp.zeros((BPG, 1), jnp.float32)
    left = jnp.concatenate([z1, count[:, :-1]], axis=1)
    right = jnp.concatenate([count[:, 1:], z1], axis=1)
    zf = jnp.zeros((BPG, PATCH_NUM), jnp.float32)
    h = count * 2.0 + jnp.where(col > 0, left, zf) + jnp.where(col < GRID_W - 1, right, zf)
    zr = jnp.zeros((BPG, GRID_W), jnp.float32)
    up = jnp.concatenate([zr, h[:, :-GRID_W]], axis=1)
    down = jnp.concatenate([h[:, GRID_W:], zr], axis=1)
    sm = h * 2.0 + up + down

    lane = lax.broadcasted_iota(jnp.int32, (BPG, PATCH_NUM), 1).astype(jnp.float32)
    key = sm * 1024.0 + (1023.0 - lane)
    keyT = key.T

    ones_col = jnp.ones((PATCH_NUM, 1), jnp.float32)
    riota = lax.broadcasted_iota(jnp.int32, (PATCH_NUM, SELECT_NUM), 1).astype(jnp.float32)
    widx = lax.broadcasted_iota(jnp.int32, (1, PATCH_NUM), 1).astype(jnp.float32) + 1.0
    for sidx in range(BPG):
        krow = key[sidx:sidx + 1, :]
        kcol = keyT[:, sidx:sidx + 1]
        g = (krow > kcol).astype(jnp.float32)
        rank = lax.dot_general(g, ones_col, (((1,), (0,)), ((), ())),
                               preferred_element_type=jnp.float32)
        oh = (rank == riota).astype(jnp.float32)
        row = lax.dot_general(widx, oh, (((1,), (0,)), ((), ())),
                              preferred_element_type=jnp.float32)
        o_ref[sidx:sidx + 1, :] = row.astype(jnp.int32)


def kernel(x, select_num):
    score = x[:, :, 0, 1:]
    score_t = jnp.transpose(score, (1, 0, 2))
    out = pl.pallas_call(
        _body,
        grid=(B // BPG,),
        in_specs=[pl.BlockSpec((NUM_HEADS, BPG, PATCH_NUM), lambda g: (0, g, 0))],
        out_specs=pl.BlockSpec((BPG, SELECT_NUM), lambda g: (g, 0)),
        out_shape=jax.ShapeDtypeStruct((B, SELECT_NUM), jnp.int32),
    )(score_t)
    return out + (select_num - SELECT_NUM)


# speedup vs baseline: 2.6302x; 1.0981x over previous
import jax
import jax.numpy as jnp
from jax import lax
from jax.experimental import pallas as pl
from jax.experimental.pallas import tpu as pltpu
from jax.experimental.pallas import tpu_sc as plsc

B = 32
NUM_HEADS = 12
PATCH_NUM = 576
VOTE_PERHEAD = 24
SELECT_NUM = 128
GRID_W = 24
NVREG = PATCH_NUM // 16
MSLOTS = 3
PAD = 24
NEG_INF = float("-inf")
INT_MIN = -2147483648


def _sc_body(score_hbm, out_hbm, s_ref, m_ref, hist_ref, hbuf_ref, kbuf_ref,
             m2_ref, obuf_ref):
    wid = lax.axis_index("s") * 2 + lax.axis_index("c")
    pltpu.sync_copy(score_hbm.at[pl.ds(wid * (NUM_HEADS * PATCH_NUM),
                                       NUM_HEADS * PATCH_NUM)], s_ref)

    i0 = lax.iota(jnp.int32, 16)
    zeros16 = jnp.zeros((16,), jnp.float32)
    ones16 = jnp.ones((16,), jnp.float32)

    for k in range((PATCH_NUM + 2 * PAD) // 16):
        hist_ref[pl.ds(16 * k, 16)] = zeros16
        hbuf_ref[pl.ds(16 * k, 16)] = zeros16

    for h in range(NUM_HEADS):
        for k in range(MSLOTS):
            acc = jnp.full((16,), NEG_INF, jnp.float32)
            for l in range(16):
                j = 16 * k + l
                if j < NVREG:
                    mv = jnp.max(s_ref[pl.ds(PATCH_NUM * h + 16 * j, 16)])
                    acc = jnp.where(i0 == l, mv, acc)
            m_ref[pl.ds(48 * h + 16 * k, 16)] = acc

    def _extract_round(_, carry):
        for h in range(NUM_HEADS):
            m0 = m_ref[pl.ds(48 * h, 16)]
            m1 = m_ref[pl.ds(48 * h + 16, 16)]
            m2 = m_ref[pl.ds(48 * h + 32, 16)]
            mval = jnp.max(jnp.maximum(jnp.maximum(m0, m1), m2))
            jv = jnp.minimum(
                jnp.minimum(jnp.where(m0 == mval, i0, 64),
                            jnp.where(m1 == mval, i0 + 16, 64)),
                jnp.where(m2 == mval, i0 + 32, 64))
            j = jnp.min(jv)
            v = s_ref[pl.ds(PATCH_NUM * h + 16 * j, 16)]
            lane = plsc.all_reduce_ffs(v == mval)
            onehot = i0 == lane
            hw = hist_ref[pl.ds(PAD + 16 * j, 16)]
            hist_ref[pl.ds(PAD + 16 * j, 16)] = hw + jnp.where(onehot, ones16, zeros16)
            v = jnp.where(onehot, NEG_INF, v)
            s_ref[pl.ds(PATCH_NUM * h + 16 * j, 16)] = v
            newmax = jnp.max(v)
            mslot = m_ref[pl.ds(48 * h + 16 * (j // 16), 16)]
            m_ref[pl.ds(48 * h + 16 * (j // 16), 16)] = jnp.where(
                i0 == j % 16, newmax, mslot)
        return carry

    lax.fori_loop(0, VOTE_PERHEAD, _extract_round, 0)

    for j in range(NVREG):
        col = (i0 + 16 * j) % GRID_W
        c = hist_ref[pl.ds(PAD + 16 * j, 16)]
        lft = hist_ref[pl.ds(PAD + 16 * j - 1, 16)]
        rgt = hist_ref[pl.ds(PAD + 16 * j + 1, 16)]
        hbuf_ref[pl.ds(PAD + 16 * j, 16)] = (
            c * 2.0 + jnp.where(col > 0, lft, zeros16)
            + jnp.where(col < GRID_W - 1, rgt, zeros16))
    for j in range(NVREG):
        hc = hbuf_ref[pl.ds(PAD + 16 * j, 16)]
        up = hbuf_ref[pl.ds(16 * j, 16)]
        dn = hbuf_ref[pl.ds(2 * PAD + 16 * j, 16)]
        sm = (hc * 2.0 + up + dn).astype(jnp.int32)
        kbuf_ref[pl.ds(16 * j, 16)] = sm * 1024 + (1023 - (i0 + 16 * j))

    for k in range(MSLOTS):
        acc2 = jnp.full((16,), INT_MIN, jnp.int32)
        for l in range(16):
            j = 16 * k + l
            if j < NVREG:
                mv2 = jnp.max(kbuf_ref[pl.ds(16 * j, 16)])
                acc2 = jnp.where(i0 == l, mv2, acc2)
        m2_ref[pl.ds(16 * k, 16)] = acc2

    def _select_round(r, carry):
        q0 = m2_ref[pl.ds(0, 16)]
        q1 = m2_ref[pl.ds(16, 16)]
        q2 = m2_ref[pl.ds(32, 16)]
        mk = jnp.max(jnp.maximum(jnp.maximum(q0, q1), q2))
        p = 1023 - (mk & 1023)
        j = p // 16
        v = kbuf_ref[pl.ds(16 * j, 16)]
        v = jnp.where(i0 == p % 16, INT_MIN, v)
        kbuf_ref[pl.ds(16 * j, 16)] = v
        newmax = jnp.max(v)
        qslot = m2_ref[pl.ds(16 * (j // 16), 16)]
        m2_ref[pl.ds(16 * (j // 16), 16)] = jnp.where(i0 == j % 16, newmax, qslot)
        ow = obuf_ref[pl.ds(16 * (r // 16), 16)]
        obuf_ref[pl.ds(16 * (r // 16), 16)] = jnp.where(i0 == r % 16, p + 1, ow)
        return carry

    lax.fori_loop(0, SELECT_NUM, _select_round, 0)

    pltpu.sync_copy(obuf_ref, out_hbm.at[pl.ds(wid * SELECT_NUM, SELECT_NUM)])


def kernel(x, select_num):
    score = x[:, :, 0, 1:].reshape(B * NUM_HEADS * PATCH_NUM)
    fn = pl.kernel(
        _sc_body,
        out_type=jax.ShapeDtypeStruct((B * SELECT_NUM,), jnp.int32),
        mesh=plsc.VectorSubcoreMesh(core_axis_name="c", subcore_axis_name="s",
                                    num_cores=2, num_subcores=16),
        scratch_types=[
            pltpu.VMEM((NUM_HEADS * PATCH_NUM,), jnp.float32),
            pltpu.VMEM((48 * NUM_HEADS,), jnp.float32),
            pltpu.VMEM((PATCH_NUM + 2 * PAD,), jnp.float32),
            pltpu.VMEM((PATCH_NUM + 2 * PAD,), jnp.float32),
            pltpu.VMEM((PATCH_NUM,), jnp.int32),
            pltpu.VMEM((48,), jnp.int32),
            pltpu.VMEM((SELECT_NUM,), jnp.int32),
        ],
        compiler_params=pltpu.CompilerParams(needs_layout_passes=False),
    )
    out = fn(score)
    return out.reshape(B, SELECT_NUM) + (select_num - SELECT_NUM)


# speedup vs baseline: 3.0506x; 1.1598x over previous
import jax
import jax.numpy as jnp
from jax import lax
from jax.experimental import pallas as pl
from jax.experimental.pallas import tpu as pltpu
from jax.experimental.pallas import tpu_sc as plsc

B = 32
NUM_HEADS = 12
PATCH_NUM = 576
VOTE_PERHEAD = 24
SELECT_NUM = 128
GRID_W = 24
NVREG = PATCH_NUM // 16
MSLOTS = 3
PAD = 24
NEG_INF = float("-inf")
INT_MIN = -2147483648


def _sc_body(score_hbm, out_hbm, s_ref, m_ref, hist_ref, hbuf_ref, cbuf_ref,
             hcnt_ref, scnt_ref, obuf_ref):
    wid = lax.axis_index("s") * 2 + lax.axis_index("c")
    pltpu.sync_copy(score_hbm.at[pl.ds(wid * (NUM_HEADS * PATCH_NUM),
                                       NUM_HEADS * PATCH_NUM)], s_ref)

    i0 = lax.iota(jnp.int32, 16)
    zeros16 = jnp.zeros((16,), jnp.float32)
    ones16 = jnp.ones((16,), jnp.float32)

    for k in range((PATCH_NUM + 2 * PAD) // 16):
        hist_ref[pl.ds(16 * k, 16)] = zeros16
        hbuf_ref[pl.ds(16 * k, 16)] = zeros16

    for h in range(NUM_HEADS):
        for k in range(MSLOTS):
            nvalid = min(16, NVREG - 16 * k)
            lane_v = i0 if nvalid == 16 else jnp.minimum(i0, nvalid - 1)
            base = PATCH_NUM * h + 256 * k + 16 * lane_v
            acc = plsc.load_gather(s_ref, [base])
            for t in range(1, 16):
                acc = jnp.maximum(acc, plsc.load_gather(s_ref, [base + t]))
            if nvalid < 16:
                acc = jnp.where(i0 < nvalid, acc, jnp.full((16,), NEG_INF, jnp.float32))
            m_ref[pl.ds(48 * h + 16 * k, 16)] = acc

    def _extract_round(_, carry):
        mslots = [[m_ref[pl.ds(48 * h + 16 * k, 16)] for k in range(MSLOTS)]
                  for h in range(NUM_HEADS)]
        mvals = [jnp.max(jnp.maximum(jnp.maximum(m[0], m[1]), m[2]))
                 for m in mslots]
        js = []
        for h in range(NUM_HEADS):
            m, mval = mslots[h], mvals[h]
            f0 = plsc.all_reduce_ffs(m[0] == mval)
            f1 = plsc.all_reduce_ffs(m[1] == mval)
            f2 = plsc.all_reduce_ffs(m[2] == mval)
            jv = jnp.minimum(
                jnp.minimum(jnp.where(f0 == 16, 64, f0),
                            jnp.where(f1 == 16, 64, f1 + 16)),
                f2 + 32)
            js.append(jv[0])
        vs = [s_ref[pl.ds(PATCH_NUM * h + 16 * js[h], 16)]
              for h in range(NUM_HEADS)]
        onehots = []
        newmaxs = []
        for h in range(NUM_HEADS):
            onehot = i0 == plsc.all_reduce_ffs(vs[h] == mvals[h])
            vnew = jnp.where(onehot, NEG_INF, vs[h])
            s_ref[pl.ds(PATCH_NUM * h + 16 * js[h], 16)] = vnew
            onehots.append(onehot)
            newmaxs.append(jnp.max(vnew))
        for h in range(NUM_HEADS):
            j = js[h]
            slot = 48 * h + ((j >> 4) << 4)
            mslot = m_ref[pl.ds(slot, 16)]
            m_ref[pl.ds(slot, 16)] = jnp.where(
                i0 == (j & 15), newmaxs[h], mslot)
        for h in range(NUM_HEADS):
            hw = hist_ref[pl.ds(PAD + 16 * js[h], 16)]
            hist_ref[pl.ds(PAD + 16 * js[h], 16)] = hw + jnp.where(
                onehots[h], ones16, zeros16)
        return carry

    lax.fori_loop(0, VOTE_PERHEAD, _extract_round, 0)

    cols = [(i0 + 16 * m) % GRID_W for m in range(3)]
    for j in range(NVREG):
        col = cols[j % 3]
        c = hist_ref[pl.ds(PAD + 16 * j, 16)]
        lft = hist_ref[pl.ds(PAD + 16 * j - 1, 16)]
        rgt = hist_ref[pl.ds(PAD + 16 * j + 1, 16)]
        hbuf_ref[pl.ds(PAD + 16 * j, 16)] = (
            c * 2.0 + jnp.where(col > 0, lft, zeros16)
            + jnp.where(col < GRID_W - 1, rgt, zeros16))
    for j in range(NVREG):
        hc = hbuf_ref[pl.ds(PAD + 16 * j, 16)]
        up = hbuf_ref[pl.ds(16 * j, 16)]
        dn = hbuf_ref[pl.ds(2 * PAD + 16 * j, 16)]
        cbuf_ref[pl.ds(16 * j, 16)] = (hc * 2.0 + up + dn).astype(jnp.int32)

    calib = plsc.scan_count(jnp.zeros((16,), jnp.int32))[0] - i0
    NB = 208 // 16
    zi16 = jnp.zeros((16,), jnp.int32)
    for k in range(NB):
        hcnt_ref[pl.ds(16 * k, 16)] = zi16
    for j in range(NVREG):
        c = cbuf_ref[pl.ds(16 * j, 16)]
        occ, lastm = plsc.scan_count(c)
        plsc.addupdate_scatter(hcnt_ref, [c], occ - calib + 1, mask=lastm)
    carry = jnp.int32(0)
    for k in reversed(range(NB)):
        hv = hcnt_ref[pl.ds(16 * k, 16)]
        tk = lax.rev(plsc.cumsum(lax.rev(hv, (0,))), (0,)) + carry
        scnt_ref[pl.ds(16 * k, 16)] = tk - hv
        carry = tk[0]
    for j in range(NVREG):
        c = cbuf_ref[pl.ds(16 * j, 16)]
        base = plsc.load_gather(scnt_ref, [c])
        occ, lastm = plsc.scan_count(c)
        occ0 = occ - calib
        plsc.store_scatter(obuf_ref, [base + occ0], i0 + (16 * j + 1))
        plsc.addupdate_scatter(scnt_ref, [c], occ0 + 1, mask=lastm)

    pltpu.sync_copy(obuf_ref.at[pl.ds(0, SELECT_NUM)],
                    out_hbm.at[pl.ds(wid * SELECT_NUM, SELECT_NUM)])


def kernel(x, select_num):
    score = x[:, :, 0, 1:].reshape(B * NUM_HEADS * PATCH_NUM)
    fn = pl.kernel(
        _sc_body,
        out_type=jax.ShapeDtypeStruct((B * SELECT_NUM,), jnp.int32),
        mesh=plsc.VectorSubcoreMesh(core_axis_name="c", subcore_axis_name="s",
                                    num_cores=2, num_subcores=16),
        scratch_types=[
            pltpu.VMEM((NUM_HEADS * PATCH_NUM,), jnp.float32),
            pltpu.VMEM((48 * NUM_HEADS,), jnp.float32),
            pltpu.VMEM((PATCH_NUM + 2 * PAD,), jnp.float32),
            pltpu.VMEM((PATCH_NUM + 2 * PAD,), jnp.float32),
            pltpu.VMEM((PATCH_NUM,), jnp.int32),
            pltpu.VMEM((208,), jnp.int32),
            pltpu.VMEM((208,), jnp.int32),
            pltpu.VMEM((PATCH_NUM,), jnp.int32),
        ],
        compiler_params=pltpu.CompilerParams(needs_layout_passes=False),
    )
    out = fn(score)
    return out.reshape(B, SELECT_NUM) + (select_num - SELECT_NUM)


# speedup vs baseline: 3.1473x; 1.0317x over previous
import jax
import jax.numpy as jnp
from jax import lax
from jax.experimental import pallas as pl
from jax.experimental.pallas import tpu as pltpu
from jax.experimental.pallas import tpu_sc as plsc

B = 32
NUM_HEADS = 12
PATCH_NUM = 576
VOTE_PERHEAD = 24
SELECT_NUM = 128
GRID_W = 24
NVREG = PATCH_NUM // 16
MSLOTS = 3
PAD = 24
NEG_INF = float("-inf")
INT_MIN = -2147483648


def _sc_body(score_hbm, out_hbm, s_ref, m_ref, hist_ref, hbuf_ref, cbuf_ref,
             hcnt_ref, scnt_ref, obuf_ref):
    wid = lax.axis_index("s") * 2 + lax.axis_index("c")
    pltpu.sync_copy(score_hbm.at[pl.ds(wid * (NUM_HEADS * PATCH_NUM),
                                       NUM_HEADS * PATCH_NUM)], s_ref)

    i0 = lax.iota(jnp.int32, 16)
    zeros16 = jnp.zeros((16,), jnp.float32)
    ones16 = jnp.ones((16,), jnp.float32)

    for k in range((PATCH_NUM + 2 * PAD) // 16):
        hist_ref[pl.ds(16 * k, 16)] = zeros16
        hbuf_ref[pl.ds(16 * k, 16)] = zeros16

    hbase = PATCH_NUM * jnp.minimum(i0, NUM_HEADS - 1)

    def _init_vreg(j, carry):
        base = hbase + 16 * j
        acc = plsc.load_gather(s_ref, [base])
        for t in range(1, 16):
            acc = jnp.maximum(acc, plsc.load_gather(s_ref, [base + t]))
        m_ref[pl.ds(16 * j, 16)] = acc
        return carry

    lax.fori_loop(0, NVREG, _init_vreg, 0)

    def _extract_round(_, carry):
        mts = [m_ref[pl.ds(16 * j, 16)] for j in range(NVREG)]
        mvec = mts[0]
        for j in range(1, NVREG):
            mvec = jnp.maximum(mvec, mts[j])
        jvec = jnp.full((16,), 64, jnp.int32)
        for j in range(NVREG - 1, -1, -1):
            jvec = jnp.where(mts[j] == mvec, j, jvec)
        js = [jvec[h] for h in range(NUM_HEADS)]
        mvals = [mvec[h] for h in range(NUM_HEADS)]
        vs = [s_ref[pl.ds(PATCH_NUM * h + 16 * js[h], 16)]
              for h in range(NUM_HEADS)]
        onehots = []
        for h in range(NUM_HEADS):
            onehot = i0 == plsc.all_reduce_ffs(vs[h] == mvals[h])
            vnew = jnp.where(onehot, NEG_INF, vs[h])
            s_ref[pl.ds(PATCH_NUM * h + 16 * js[h], 16)] = vnew
            onehots.append(onehot)
            mt = m_ref[pl.ds(16 * js[h], 16)]
            m_ref[pl.ds(16 * js[h], 16)] = jnp.where(i0 == h, jnp.max(vnew), mt)
        for h in range(NUM_HEADS):
            hw = hist_ref[pl.ds(PAD + 16 * js[h], 16)]
            hist_ref[pl.ds(PAD + 16 * js[h], 16)] = hw + jnp.where(
                onehots[h], ones16, zeros16)
        return carry

    lax.fori_loop(0, VOTE_PERHEAD, _extract_round, 0)

    cols = [(i0 + 16 * m) % GRID_W for m in range(3)]
    for j in range(NVREG):
        col = cols[j % 3]
        c = hist_ref[pl.ds(PAD + 16 * j, 16)]
        lft = hist_ref[pl.ds(PAD + 16 * j - 1, 16)]
        rgt = hist_ref[pl.ds(PAD + 16 * j + 1, 16)]
        hbuf_ref[pl.ds(PAD + 16 * j, 16)] = (
            c * 2.0 + jnp.where(col > 0, lft, zeros16)
            + jnp.where(col < GRID_W - 1, rgt, zeros16))
    for j in range(NVREG):
        hc = hbuf_ref[pl.ds(PAD + 16 * j, 16)]
        up = hbuf_ref[pl.ds(16 * j, 16)]
        dn = hbuf_ref[pl.ds(2 * PAD + 16 * j, 16)]
        cbuf_ref[pl.ds(16 * j, 16)] = (hc * 2.0 + up + dn).astype(jnp.int32)

    calib = plsc.scan_count(jnp.zeros((16,), jnp.int32))[0] - i0
    NB = 208 // 16
    zi16 = jnp.zeros((16,), jnp.int32)
    for k in range(NB):
        hcnt_ref[pl.ds(16 * k, 16)] = zi16
    for j in range(NVREG):
        c = cbuf_ref[pl.ds(16 * j, 16)]
        occ, lastm = plsc.scan_count(c)
        plsc.addupdate_scatter(hcnt_ref, [c], occ - calib + 1, mask=lastm)
    carry = jnp.int32(0)
    for k in reversed(range(NB)):
        hv = hcnt_ref[pl.ds(16 * k, 16)]
        tk = lax.rev(plsc.cumsum(lax.rev(hv, (0,))), (0,)) + carry
        scnt_ref[pl.ds(16 * k, 16)] = tk - hv
        carry = tk[0]
    for j in range(NVREG):
        c = cbuf_ref[pl.ds(16 * j, 16)]
        base = plsc.load_gather(scnt_ref, [c])
        occ, lastm = plsc.scan_count(c)
        occ0 = occ - calib
        plsc.store_scatter(obuf_ref, [base + occ0], i0 + (16 * j + 1))
        plsc.addupdate_scatter(scnt_ref, [c], occ0 + 1, mask=lastm)

    pltpu.sync_copy(obuf_ref.at[pl.ds(0, SELECT_NUM)],
                    out_hbm.at[pl.ds(wid * SELECT_NUM, SELECT_NUM)])


def kernel(x, select_num):
    score = x[:, :, 0, 1:].reshape(B * NUM_HEADS * PATCH_NUM)
    fn = pl.kernel(
        _sc_body,
        out_type=jax.ShapeDtypeStruct((B * SELECT_NUM,), jnp.int32),
        mesh=plsc.VectorSubcoreMesh(core_axis_name="c", subcore_axis_name="s",
                                    num_cores=2, num_subcores=16),
        scratch_types=[
            pltpu.VMEM((NUM_HEADS * PATCH_NUM,), jnp.float32),
            pltpu.VMEM((48 * NUM_HEADS,), jnp.float32),
            pltpu.VMEM((PATCH_NUM + 2 * PAD,), jnp.float32),
            pltpu.VMEM((PATCH_NUM + 2 * PAD,), jnp.float32),
            pltpu.VMEM((PATCH_NUM,), jnp.int32),
            pltpu.VMEM((208,), jnp.int32),
            pltpu.VMEM((208,), jnp.int32),
            pltpu.VMEM((PATCH_NUM,), jnp.int32),
        ],
        compiler_params=pltpu.CompilerParams(needs_layout_passes=False),
    )
    out = fn(score)
    return out.reshape(B, SELECT_NUM) + (select_num - SELECT_NUM)


# speedup vs baseline: 3.2048x; 1.0183x over previous
import jax
import jax.numpy as jnp
from jax import lax
from jax.experimental import pallas as pl
from jax.experimental.pallas import tpu as pltpu
from jax.experimental.pallas import tpu_sc as plsc

B = 32
NUM_HEADS = 12
PATCH_NUM = 576
VOTE_PERHEAD = 24
SELECT_NUM = 128
GRID_W = 24
NVREG = PATCH_NUM // 16
MSLOTS = 3
PAD = 24
NEG_INF = float("-inf")
INT_MIN = -2147483648


def _sc_body(score_hbm, out_hbm, s_ref, m_ref, hist_ref, hbuf_ref, cbuf_ref,
             hcnt_ref, scnt_ref, obuf_ref):
    wid = lax.axis_index("s") * 2 + lax.axis_index("c")
    pltpu.sync_copy(score_hbm.at[pl.ds(wid * (NUM_HEADS * PATCH_NUM),
                                       NUM_HEADS * PATCH_NUM)], s_ref)

    i0 = lax.iota(jnp.int32, 16)
    zeros16 = jnp.zeros((16,), jnp.float32)
    ones16 = jnp.ones((16,), jnp.float32)

    for a in (0, 16, PATCH_NUM + PAD - 8, PATCH_NUM + 2 * PAD - 16):
        hist_ref[pl.ds(a, 16)] = zeros16
        hbuf_ref[pl.ds(a, 16)] = zeros16

    hbase = PATCH_NUM * jnp.minimum(i0, NUM_HEADS - 1)

    def _init_vreg(j, carry):
        base = hbase + 16 * j
        acc = plsc.load_gather(s_ref, [base])
        for t in range(1, 16):
            acc = jnp.maximum(acc, plsc.load_gather(s_ref, [base + t]))
        m_ref[pl.ds(16 * j, 16)] = acc
        return carry

    lax.fori_loop(0, NVREG, _init_vreg, 0)

    def _extract_round(_, carry):
        mts = [m_ref[pl.ds(16 * j, 16)] for j in range(NVREG)]
        mvec = mts[0]
        for j in range(1, NVREG):
            mvec = jnp.maximum(mvec, mts[j])
        jvec = jnp.full((16,), 64, jnp.int32)
        for j in range(NVREG - 1, -1, -1):
            jvec = jnp.where(mts[j] == mvec, j, jvec)
        js = [jvec[h] for h in range(NUM_HEADS)]
        mvals = [mvec[h] for h in range(NUM_HEADS)]
        vs = [s_ref[pl.ds(PATCH_NUM * h + 16 * js[h], 16)]
              for h in range(NUM_HEADS)]
        for h in range(NUM_HEADS):
            onehot = i0 == plsc.all_reduce_ffs(vs[h] == mvals[h])
            vnew = jnp.where(onehot, NEG_INF, vs[h])
            s_ref[pl.ds(PATCH_NUM * h + 16 * js[h], 16)] = vnew
            mt = m_ref[pl.ds(16 * js[h], 16)]
            m_ref[pl.ds(16 * js[h], 16)] = jnp.where(i0 == h, jnp.max(vnew), mt)
        return carry

    lax.fori_loop(0, VOTE_PERHEAD, _extract_round, 0)

    def _count_votes(j, carry):
        cnt = jnp.zeros((16,), jnp.float32)
        for h in range(NUM_HEADS):
            v = s_ref[pl.ds(PATCH_NUM * h + 16 * j, 16)]
            cnt = cnt + jnp.where(v == NEG_INF, ones16, zeros16)
        hist_ref[pl.ds(PAD + 16 * j, 16)] = cnt
        return carry

    lax.fori_loop(0, NVREG, _count_votes, 0)

    cols = [(i0 + 16 * m) % GRID_W for m in range(3)]
    for j in range(NVREG):
        col = cols[j % 3]
        c = hist_ref[pl.ds(PAD + 16 * j, 16)]
        lft = hist_ref[pl.ds(PAD + 16 * j - 1, 16)]
        rgt = hist_ref[pl.ds(PAD + 16 * j + 1, 16)]
        hbuf_ref[pl.ds(PAD + 16 * j, 16)] = (
            c * 2.0 + jnp.where(col > 0, lft, zeros16)
            + jnp.where(col < GRID_W - 1, rgt, zeros16))
    for j in range(NVREG):
        hc = hbuf_ref[pl.ds(PAD + 16 * j, 16)]
        up = hbuf_ref[pl.ds(16 * j, 16)]
        dn = hbuf_ref[pl.ds(2 * PAD + 16 * j, 16)]
        cbuf_ref[pl.ds(16 * j, 16)] = (hc * 2.0 + up + dn).astype(jnp.int32)

    calib = plsc.scan_count(jnp.zeros((16,), jnp.int32))[0] - i0
    NB = 208 // 16
    zi16 = jnp.zeros((16,), jnp.int32)
    for k in range(NB):
        hcnt_ref[pl.ds(16 * k, 16)] = zi16
    for j in range(NVREG):
        c = cbuf_ref[pl.ds(16 * j, 16)]
        occ, lastm = plsc.scan_count(c)
        plsc.addupdate_scatter(hcnt_ref, [c], occ - calib + 1, mask=lastm)
    carry = jnp.int32(0)
    for k in reversed(range(NB)):
        hv = hcnt_ref[pl.ds(16 * k, 16)]
        tk = lax.rev(plsc.cumsum(lax.rev(hv, (0,))), (0,)) + carry
        scnt_ref[pl.ds(16 * k, 16)] = tk - hv
        carry = tk[0]
    for j in range(NVREG):
        c = cbuf_ref[pl.ds(16 * j, 16)]
        base = plsc.load_gather(scnt_ref, [c])
        occ, lastm = plsc.scan_count(c)
        occ0 = occ - calib
        plsc.store_scatter(obuf_ref, [base + occ0], i0 + (16 * j + 1))
        plsc.addupdate_scatter(scnt_ref, [c], occ0 + 1, mask=lastm)

    pltpu.sync_copy(obuf_ref.at[pl.ds(0, SELECT_NUM)],
                    out_hbm.at[pl.ds(wid * SELECT_NUM, SELECT_NUM)])


def kernel(x, select_num):
    score = x[:, :, 0, 1:].reshape(B * NUM_HEADS * PATCH_NUM)
    fn = pl.kernel(
        _sc_body,
        out_type=jax.ShapeDtypeStruct((B * SELECT_NUM,), jnp.int32),
        mesh=plsc.VectorSubcoreMesh(core_axis_name="c", subcore_axis_name="s",
                                    num_cores=2, num_subcores=16),
        scratch_types=[
            pltpu.VMEM((NUM_HEADS * PATCH_NUM,), jnp.float32),
            pltpu.VMEM((48 * NUM_HEADS,), jnp.float32),
            pltpu.VMEM((PATCH_NUM + 2 * PAD,), jnp.float32),
            pltpu.VMEM((PATCH_NUM + 2 * PAD,), jnp.float32),
            pltpu.VMEM((PATCH_NUM,), jnp.int32),
            pltpu.VMEM((208,), jnp.int32),
            pltpu.VMEM((208,), jnp.int32),
            pltpu.VMEM((PATCH_NUM,), jnp.int32),
        ],
        compiler_params=pltpu.CompilerParams(needs_layout_passes=False),
    )
    out = fn(score)
    return out.reshape(B, SELECT_NUM) + (select_num - SELECT_NUM)


# speedup vs baseline: 3.3939x; 1.0590x over previous
import jax
import jax.numpy as jnp
from jax import lax
from jax.experimental import pallas as pl
from jax.experimental.pallas import tpu as pltpu
from jax.experimental.pallas import tpu_sc as plsc

B = 32
NUM_HEADS = 12
PATCH_NUM = 576
VOTE_PERHEAD = 24
SELECT_NUM = 128
GRID_W = 24
NVREG = PATCH_NUM // 16
MSLOTS = 3
PAD = 24
NEG_INF = float("-inf")
INT_MIN = -2147483648


def _sc_body(score_hbm, out_hbm, s_ref, m_ref, hist_ref, hbuf_ref, cbuf_ref,
             hcnt_ref, scnt_ref, obuf_ref):
    wid = lax.axis_index("s") * 2 + lax.axis_index("c")
    pltpu.sync_copy(score_hbm.at[pl.ds(wid * (NUM_HEADS * PATCH_NUM),
                                       NUM_HEADS * PATCH_NUM)], s_ref)

    i0 = lax.iota(jnp.int32, 16)
    zeros16 = jnp.zeros((16,), jnp.float32)
    ones16 = jnp.ones((16,), jnp.float32)

    for a in (0, 16, PATCH_NUM + PAD - 8, PATCH_NUM + 2 * PAD - 16):
        hist_ref[pl.ds(a, 16)] = zeros16
        hbuf_ref[pl.ds(a, 16)] = zeros16

    hbase = PATCH_NUM * jnp.minimum(i0, NUM_HEADS - 1)

    def _init_vreg(j, carry):
        base = hbase + 16 * j
        acc = plsc.load_gather(s_ref, [base])
        for t in range(1, 16):
            acc = jnp.maximum(acc, plsc.load_gather(s_ref, [base + t]))
        m_ref[pl.ds(16 * j, 16)] = acc
        return carry

    lax.fori_loop(0, NVREG, _init_vreg, 0)

    def _extract_round(_, carry):
        mts = [m_ref[pl.ds(16 * j, 16)] for j in range(NVREG)]
        mvec = mts[0]
        for j in range(1, NVREG):
            mvec = jnp.maximum(mvec, mts[j])
        jvec = jnp.full((16,), 64, jnp.int32)
        for j in range(NVREG - 1, -1, -1):
            jvec = jnp.where(mts[j] == mvec, j, jvec)
        js = [jvec[h] for h in range(NUM_HEADS)]
        mvals = [mvec[h] for h in range(NUM_HEADS)]
        vs = [s_ref[pl.ds(PATCH_NUM * h + 16 * js[h], 16)]
              for h in range(NUM_HEADS)]
        nmv = zeros16
        for h in range(NUM_HEADS):
            onehot = i0 == plsc.all_reduce_ffs(vs[h] == mvals[h])
            vnew = jnp.where(onehot, NEG_INF, vs[h])
            s_ref[pl.ds(PATCH_NUM * h + 16 * js[h], 16)] = vnew
            nmv = jnp.where(i0 == h, jnp.max(vnew), nmv)
        plsc.store_scatter(m_ref, [16 * jnp.minimum(jvec, NVREG - 1) + i0], nmv,
                           mask=i0 < NUM_HEADS)
        return carry

    lax.fori_loop(0, VOTE_PERHEAD, _extract_round, 0)

    def _count_votes(j, carry):
        cnt = jnp.zeros((16,), jnp.float32)
        for h in range(NUM_HEADS):
            v = s_ref[pl.ds(PATCH_NUM * h + 16 * j, 16)]
            cnt = cnt + jnp.where(v == NEG_INF, ones16, zeros16)
        hist_ref[pl.ds(PAD + 16 * j, 16)] = cnt
        return carry

    lax.fori_loop(0, NVREG, _count_votes, 0)

    cols = [(i0 + 16 * m) % GRID_W for m in range(3)]
    for j in range(NVREG):
        col = cols[j % 3]
        c = hist_ref[pl.ds(PAD + 16 * j, 16)]
        lft = hist_ref[pl.ds(PAD + 16 * j - 1, 16)]
        rgt = hist_ref[pl.ds(PAD + 16 * j + 1, 16)]
        hbuf_ref[pl.ds(PAD + 16 * j, 16)] = (
            c * 2.0 + jnp.where(col > 0, lft, zeros16)
            + jnp.where(col < GRID_W - 1, rgt, zeros16))
    for j in range(NVREG):
        hc = hbuf_ref[pl.ds(PAD + 16 * j, 16)]
        up = hbuf_ref[pl.ds(16 * j, 16)]
        dn = hbuf_ref[pl.ds(2 * PAD + 16 * j, 16)]
        cbuf_ref[pl.ds(16 * j, 16)] = (hc * 2.0 + up + dn).astype(jnp.int32)

    calib = plsc.scan_count(jnp.zeros((16,), jnp.int32))[0] - i0
    NB = 208 // 16
    zi16 = jnp.zeros((16,), jnp.int32)
    for k in range(NB):
        hcnt_ref[pl.ds(16 * k, 16)] = zi16
    for j in range(NVREG):
        c = cbuf_ref[pl.ds(16 * j, 16)]
        occ, lastm = plsc.scan_count(c)
        plsc.addupdate_scatter(hcnt_ref, [c], occ - calib + 1, mask=lastm)
    carry = jnp.int32(0)
    for k in reversed(range(NB)):
        hv = hcnt_ref[pl.ds(16 * k, 16)]
        tk = lax.rev(plsc.cumsum(lax.rev(hv, (0,))), (0,)) + carry
        scnt_ref[pl.ds(16 * k, 16)] = tk - hv
        carry = tk[0]
    for j in range(NVREG):
        c = cbuf_ref[pl.ds(16 * j, 16)]
        base = plsc.load_gather(scnt_ref, [c])
        occ, lastm = plsc.scan_count(c)
        occ0 = occ - calib
        plsc.store_scatter(obuf_ref, [base + occ0], i0 + (16 * j + 1))
        plsc.addupdate_scatter(scnt_ref, [c], occ0 + 1, mask=lastm)

    pltpu.sync_copy(obuf_ref.at[pl.ds(0, SELECT_NUM)],
                    out_hbm.at[pl.ds(wid * SELECT_NUM, SELECT_NUM)])


def kernel(x, select_num):
    score = x[:, :, 0, 1:].reshape(B * NUM_HEADS * PATCH_NUM)
    fn = pl.kernel(
        _sc_body,
        out_type=jax.ShapeDtypeStruct((B * SELECT_NUM,), jnp.int32),
        mesh=plsc.VectorSubcoreMesh(core_axis_name="c", subcore_axis_name="s",
                                    num_cores=2, num_subcores=16),
        scratch_types=[
            pltpu.VMEM((NUM_HEADS * PATCH_NUM,), jnp.float32),
            pltpu.VMEM((48 * NUM_HEADS,), jnp.float32),
            pltpu.VMEM((PATCH_NUM + 2 * PAD,), jnp.float32),
            pltpu.VMEM((PATCH_NUM + 2 * PAD,), jnp.float32),
            pltpu.VMEM((PATCH_NUM,), jnp.int32),
            pltpu.VMEM((208,), jnp.int32),
            pltpu.VMEM((208,), jnp.int32),
            pltpu.VMEM((PATCH_NUM,), jnp.int32),
        ],
        compiler_params=pltpu.CompilerParams(needs_layout_passes=False),
    )
    out = fn(score)
    return out.reshape(B, SELECT_NUM) + (select_num - SELECT_NUM)


# speedup vs baseline: 3.3996x; 1.0017x over previous
import jax
import jax.numpy as jnp
from jax import lax
from jax.experimental import pallas as pl
from jax.experimental.pallas import tpu as pltpu
from jax.experimental.pallas import tpu_sc as plsc

B = 32
NUM_HEADS = 12
PATCH_NUM = 576
VOTE_PERHEAD = 24
SELECT_NUM = 128
GRID_W = 24
NVREG = PATCH_NUM // 16
PAD = 24
NEG_INF = float("-inf")


def _sc_body(score_hbm, out_hbm, s_ref, m_ref, hist_ref, hbuf_ref, cbuf_ref,
             hcnt_ref, scnt_ref, obuf_ref):
    wid = lax.axis_index("s") * 2 + lax.axis_index("c")
    pltpu.sync_copy(score_hbm.at[pl.ds(wid * (NUM_HEADS * PATCH_NUM),
                                       NUM_HEADS * PATCH_NUM)], s_ref)

    i0 = lax.iota(jnp.int32, 16)
    zeros16 = jnp.zeros((16,), jnp.float32)
    ones16 = jnp.ones((16,), jnp.float32)

    for a in (0, 16, PATCH_NUM + PAD - 8, PATCH_NUM + 2 * PAD - 16):
        hist_ref[pl.ds(a, 16)] = zeros16
        hbuf_ref[pl.ds(a, 16)] = zeros16

    hbase = PATCH_NUM * jnp.minimum(i0, NUM_HEADS - 1)

    def _init_vreg(j, carry):
        base = hbase + 16 * j
        acc = plsc.load_gather(s_ref, [base])
        for t in range(1, 16):
            acc = jnp.maximum(acc, plsc.load_gather(s_ref, [base + t]))
        m_ref[pl.ds(16 * j, 16)] = acc
        return carry

    lax.fori_loop(0, NVREG, _init_vreg, 0)

    def _extract_round(_, carry):
        mts = [m_ref[pl.ds(16 * j, 16)] for j in range(NVREG)]
        mvec = mts[0]
        for j in range(1, NVREG):
            mvec = jnp.maximum(mvec, mts[j])
        jvec = jnp.full((16,), 64, jnp.int32)
        for j in range(NVREG - 1, -1, -1):
            jvec = jnp.where(mts[j] == mvec, j, jvec)
        js = [jvec[h] for h in range(NUM_HEADS)]
        mvals = [mvec[h] for h in range(NUM_HEADS)]
        vs = [s_ref[pl.ds(PATCH_NUM * h + 16 * js[h], 16)]
              for h in range(NUM_HEADS)]
        nmv = zeros16
        for h in range(NUM_HEADS):
            onehot = i0 == plsc.all_reduce_ffs(vs[h] == mvals[h])
            vnew = jnp.where(onehot, NEG_INF, vs[h])
            s_ref[pl.ds(PATCH_NUM * h + 16 * js[h], 16)] = vnew
            nmv = jnp.where(i0 == h, jnp.max(vnew), nmv)
        plsc.store_scatter(m_ref, [16 * jnp.minimum(jvec, NVREG - 1) + i0], nmv,
                           mask=i0 < NUM_HEADS)
        return carry

    lax.fori_loop(0, VOTE_PERHEAD, _extract_round, 0)

    def _count_votes(j, carry):
        cnt = jnp.zeros((16,), jnp.float32)
        for h in range(NUM_HEADS):
            v = s_ref[pl.ds(PATCH_NUM * h + 16 * j, 16)]
            cnt = cnt + jnp.where(v == NEG_INF, ones16, zeros16)
        hist_ref[pl.ds(PAD + 16 * j, 16)] = cnt
        return carry

    lax.fori_loop(0, NVREG, _count_votes, 0)

    cols = [(i0 + 16 * m) % GRID_W for m in range(3)]
    for j in range(NVREG):
        col = cols[j % 3]
        c = hist_ref[pl.ds(PAD + 16 * j, 16)]
        lft = hist_ref[pl.ds(PAD + 16 * j - 1, 16)]
        rgt = hist_ref[pl.ds(PAD + 16 * j + 1, 16)]
        hbuf_ref[pl.ds(PAD + 16 * j, 16)] = (
            c * 2.0 + jnp.where(col > 0, lft, zeros16)
            + jnp.where(col < GRID_W - 1, rgt, zeros16))
    for j in range(NVREG):
        hc = hbuf_ref[pl.ds(PAD + 16 * j, 16)]
        up = hbuf_ref[pl.ds(16 * j, 16)]
        dn = hbuf_ref[pl.ds(2 * PAD + 16 * j, 16)]
        cbuf_ref[pl.ds(16 * j, 16)] = (hc * 2.0 + up + dn).astype(jnp.int32)

    calib = plsc.scan_count(jnp.zeros((16,), jnp.int32))[0] - i0
    NB = 208 // 16
    zi16 = jnp.zeros((16,), jnp.int32)
    for k in range(NB):
        hcnt_ref[pl.ds(16 * k, 16)] = zi16
    for j in range(NVREG):
        c = cbuf_ref[pl.ds(16 * j, 16)]
        occ, lastm = plsc.scan_count(c)
        plsc.addupdate_scatter(hcnt_ref, [c], occ - calib + 1, mask=lastm)
    carry = jnp.int32(0)
    for k in reversed(range(NB)):
        hv = hcnt_ref[pl.ds(16 * k, 16)]
        tk = lax.rev(plsc.cumsum(lax.rev(hv, (0,))), (0,)) + carry
        scnt_ref[pl.ds(16 * k, 16)] = tk - hv
        carry = tk[0]
    for j in range(NVREG):
        c = cbuf_ref[pl.ds(16 * j, 16)]
        base = plsc.load_gather(scnt_ref, [c])
        occ, lastm = plsc.scan_count(c)
        occ0 = occ - calib
        plsc.store_scatter(obuf_ref, [base + occ0], i0 + (16 * j + 1))
        plsc.addupdate_scatter(scnt_ref, [c], occ0 + 1, mask=lastm)

    pltpu.sync_copy(obuf_ref.at[pl.ds(0, SELECT_NUM)],
                    out_hbm.at[pl.ds(wid * SELECT_NUM, SELECT_NUM)])


def kernel(x, select_num):
    score = x[:, :, 0, 1:].reshape(B * NUM_HEADS * PATCH_NUM)
    fn = pl.kernel(
        _sc_body,
        out_type=jax.ShapeDtypeStruct((B * SELECT_NUM,), jnp.int32),
        mesh=plsc.VectorSubcoreMesh(core_axis_name="c", subcore_axis_name="s",
                                    num_cores=2, num_subcores=16),
        scratch_types=[
            pltpu.VMEM((NUM_HEADS * PATCH_NUM,), jnp.float32),
            pltpu.VMEM((16 * NVREG,), jnp.float32),
            pltpu.VMEM((PATCH_NUM + 2 * PAD,), jnp.float32),
            pltpu.VMEM((PATCH_NUM + 2 * PAD,), jnp.float32),
            pltpu.VMEM((PATCH_NUM,), jnp.int32),
            pltpu.VMEM((208,), jnp.int32),
            pltpu.VMEM((208,), jnp.int32),
            pltpu.VMEM((PATCH_NUM,), jnp.int32),
        ],
        compiler_params=pltpu.CompilerParams(needs_layout_passes=False),
    )
    out = fn(score)
    return out.reshape(B, SELECT_NUM) + (select_num - SELECT_NUM)
